# baseline (speedup 1.0000x reference)
import functools

import jax
import jax.numpy as jnp
from jax import lax
from jax.experimental import pallas as pl
from jax.experimental.pallas import tpu as pltpu
from jax.experimental.pallas import tpu_sc as plsc

NC = 2
NS = 16
NW = NC * NS
CHUNK = 128
NBUF = 8

_mesh = plsc.VectorSubcoreMesh(core_axis_name="c", subcore_axis_name="s")
_sc_params = pltpu.CompilerParams(use_tc_tiling_on_sc=False)


def _deg_body(dst3, zer, ones_h, out, acc, dst_v, ones_v, ss, *, cpw, rpt):
  c = lax.axis_index("c")
  s = lax.axis_index("s")
  wid = c * NS + s
  pltpu.sync_copy(zer, acc.at[pl.ds(s * rpt, rpt)])
  pltpu.sync_copy(dst3.at[wid], dst_v)
  pltpu.sync_copy(ones_h, ones_v)
  plsc.subcore_barrier()

  def body(k, carry):
    base = k * NBUF
    for b in range(NBUF):
      j = base + b
      pltpu.async_copy(ones_v, acc.at[dst_v.at[j]], ss, add=True)

      @pl.when(j >= 4)
      def _():
        pltpu.make_async_copy(ones_v, acc.at[dst_v.at[0]], ss).wait()

    return carry

  lax.fori_loop(0, cpw // NBUF, body, 0)
  for _ in range(4):
    pltpu.make_async_copy(ones_v, acc.at[dst_v.at[0]], ss).wait()
  plsc.subcore_barrier()
  pltpu.sync_copy(acc.at[pl.ds(s * rpt, rpt)], out.at[c, pl.ds(s * rpt, rpt)])


def _scat_body(g, src3, dst3, zer, out, acc, gsh, src_v, dst_v, rows_v, sg, ss,
               *, cpw, rpt):
  c = lax.axis_index("c")
  s = lax.axis_index("s")
  wid = c * NS + s
  pltpu.async_copy(g.at[pl.ds(s * rpt, rpt)], gsh.at[pl.ds(s * rpt, rpt)], sg)

  @pl.when(c == 0)
  def _():
    pltpu.async_copy(g.at[pl.ds(s * rpt, rpt)], acc.at[pl.ds(s * rpt, rpt)],
                     sg)

  @pl.when(c != 0)
  def _():
    pltpu.async_copy(zer, acc.at[pl.ds(s * rpt, rpt)], sg)

  pltpu.make_async_copy(g.at[pl.ds(s * rpt, rpt)], gsh.at[pl.ds(s * rpt, rpt)],
                        sg).wait()
  pltpu.make_async_copy(zer, acc.at[pl.ds(s * rpt, rpt)], sg).wait()
  plsc.subcore_barrier()

  rows = rows_v
  rnb = len(rows)
  ahead = 2
  depth = rnb - ahead
  hw = cpw // 4

  for phase in range(4):
    pltpu.sync_copy(src3.at[wid, pl.ds(phase * hw, hw)], src_v)
    pltpu.sync_copy(dst3.at[wid, pl.ds(phase * hw, hw)], dst_v)
    for b in range(ahead):
      pltpu.async_copy(gsh.at[src_v.at[b]], rows[b], sg)

    def body(k, carry):
      base = k * rnb
      for b in range(rnb):
        j = base + b

        @pl.when(j >= depth)
        def _():
          pltpu.make_async_copy(rows[b], acc.at[dst_v.at[0]], ss).wait()

        pltpu.make_async_copy(gsh.at[src_v.at[0]], rows[b], sg).wait()
        pltpu.async_copy(rows[b], acc.at[dst_v.at[j]], ss, add=True)

        @pl.when(j + ahead < hw)
        def _():
          pltpu.async_copy(gsh.at[src_v.at[j + ahead]],
                           rows[(b + ahead) % rnb], sg)

      return carry

    lax.fori_loop(0, hw // rnb, body, 0)
    for b in range(depth):
      pltpu.make_async_copy(rows[b], acc.at[dst_v.at[0]], ss).wait()
  plsc.subcore_barrier()
  pltpu.sync_copy(acc.at[pl.ds(s * rpt, rpt)], out.at[c, pl.ds(s * rpt, rpt)])


def _tc0_body(degpk, x, w0, dinv_out, g0):
  npk = dinv_out.shape[0]
  n = x.shape[0]
  dinv_pk = lax.rsqrt(degpk[0] + degpk[1] + 1.0)
  dinv_out[...] = dinv_pk
  xv = x[...]
  xa = xv[0:npk]
  xb = jnp.concatenate(
      [xv[npk:n], jnp.zeros((2 * npk - n, x.shape[1]), jnp.float32)], axis=0)
  t0 = jnp.concatenate(
      [jnp.dot(xa, w0[...], preferred_element_type=jnp.float32),
       jnp.dot(xb, w0[...], preferred_element_type=jnp.float32)], axis=1)
  g0[...] = dinv_pk * t0


def _tcmid_body(dinv, sp, b, w2, h_out, g_out):
  h = jnp.maximum(dinv[...] * (sp[0] + sp[1]) + b[...], 0.0)
  h_out[...] = h
  g_out[...] = dinv[...] * jnp.dot(h, w2[...],
                                   preferred_element_type=jnp.float32)


def _tclast_body(dinv, sp, b, h_out):
  h_out[...] = jnp.maximum(dinv[...] * (sp[0] + sp[1]) + b[...], 0.0)


def _tcfin_body(h1, h2, h3, h4, h5, h6, wjk1, wjk2, wjk3, wjk4, wjk5, wjk6,
                bjk, be, bo, wl1, bl1, wl2, bl2, out):
  hs = (h1, h2, h3, h4, h5, h6)
  ws = (wjk1, wjk2, wjk3, wjk4, wjk5, wjk6)
  acc = None
  for h, w in zip(hs, ws):
    t = jnp.dot(h[...], w[...], preferred_element_type=jnp.float32)
    acc = t if acc is None else acc + t
  hjk = jnp.maximum(acc + bjk[...], 0.0)
  ng = out.shape[0]
  npk = be.shape[1]
  gids = lax.broadcasted_iota(jnp.int32, (ng, npk), 0)
  ohe = jnp.where(gids == be[...], 1.0, 0.0).astype(jnp.float32)
  oho = jnp.where(gids == bo[...], 1.0, 0.0).astype(jnp.float32)
  pooled = (jnp.dot(ohe, hjk[:, 0:64], preferred_element_type=jnp.float32) +
            jnp.dot(oho, hjk[:, 64:128], preferred_element_type=jnp.float32))
  p1 = jnp.maximum(
      jnp.dot(pooled, wl1[...], preferred_element_type=jnp.float32) + bl1[...],
      0.0)
  logits = jnp.dot(p1, wl2[...], preferred_element_type=jnp.float32) + bl2[...]
  m = jnp.max(logits, axis=1, keepdims=True)
  e = jnp.exp(logits - m)
  out[...] = e / jnp.sum(e, axis=1, keepdims=True)


def _blockdiag2(w):
  a, b = w.shape
  z = jnp.zeros((a, b), jnp.float32)
  return jnp.concatenate([jnp.concatenate([w, z], axis=1),
                          jnp.concatenate([z, w], axis=1)], axis=0)


def kernel(x, edge_index, batch, params):
  n = x.shape[0]
  e = edge_index.shape[1]
  h = params['Ws'][0].shape[1]
  nl = len(params['Ws'])
  ng = 64
  ncls = params['Wl2'].shape[1]

  rpt = -(-(n + 1) // (NS * 8)) * 8
  acc_rows = NS * rpt
  npk = acc_rows // 2
  cpw = -(-e // (NW * CHUNK))
  cpw = -(-cpw // NBUF) * NBUF
  e_pad = NW * cpw * CHUNK

  def remap(v):
    return jnp.where(v < npk, 2 * v, 2 * (v - npk) + 1)

  src = remap(edge_index[0])
  dst = remap(edge_index[1])
  junk = 2 * (n - npk) + 1
  pad = e_pad - e
  src3 = jnp.concatenate([src, jnp.zeros((pad,), jnp.int32)]).reshape(
      NW, cpw, CHUNK)
  dst3 = jnp.concatenate([dst, jnp.full((pad,), junk, jnp.int32)]).reshape(
      NW, cpw, CHUNK)
  ones64 = jnp.ones((CHUNK, h), jnp.float32)
  zer64 = jnp.zeros((rpt, h), jnp.float32)

  deg_call = pl.kernel(
      functools.partial(_deg_body, cpw=cpw, rpt=rpt),
      out_type=jax.ShapeDtypeStruct((NC, acc_rows, h), jnp.float32),
      mesh=_mesh,
      scratch_types=[
          pltpu.VMEM_SHARED((acc_rows, h), jnp.float32),
          pltpu.VMEM((cpw, CHUNK), jnp.int32),
          pltpu.VMEM((CHUNK, h), jnp.float32),
          pltpu.SemaphoreType.DMA,
      ],
      compiler_params=_sc_params,
  )
  scat_call = pl.kernel(
      functools.partial(_scat_body, cpw=cpw, rpt=rpt),
      out_type=jax.ShapeDtypeStruct((NC, acc_rows, h), jnp.float32),
      mesh=_mesh,
      scratch_types=[
          pltpu.VMEM_SHARED((acc_rows, h), jnp.float32),
          pltpu.VMEM_SHARED((acc_rows, h), jnp.float32),
          pltpu.VMEM((cpw // 4, CHUNK), jnp.int32),
          pltpu.VMEM((cpw // 4, CHUNK), jnp.int32),
          [pltpu.VMEM((CHUNK, h), jnp.float32) for _ in range(5)],
          pltpu.SemaphoreType.DMA,
          pltpu.SemaphoreType.DMA,
      ],
      compiler_params=_sc_params,
  )

  tc0 = pl.pallas_call(
      _tc0_body,
      out_shape=(jax.ShapeDtypeStruct((npk, 128), jnp.float32),
                 jax.ShapeDtypeStruct((npk, 128), jnp.float32)))
  tcmid = pl.pallas_call(
      _tcmid_body,
      out_shape=(jax.ShapeDtypeStruct((npk, 128), jnp.float32),
                 jax.ShapeDtypeStruct((npk, 128), jnp.float32)))
  tclast = pl.pallas_call(
      _tclast_body, out_shape=jax.ShapeDtypeStruct((npk, 128), jnp.float32))
  tcfin = pl.pallas_call(
      _tcfin_body, out_shape=jax.ShapeDtypeStruct((ng, ncls), jnp.float32))

  w2s = [_blockdiag2(w) for w in params['Ws'][1:]]
  wjk2 = [_blockdiag2(params['Wjk'][i * h:(i + 1) * h, :]) for i in range(nl)]
  b_pk = [jnp.tile(b, 2).reshape(1, 2 * h) for b in params['bs']]
  bjk_pk = jnp.tile(params['bjk'], 2).reshape(1, 2 * h)
  bpad = jnp.full((acc_rows - n,), -1, jnp.int32)
  bfull = jnp.concatenate([batch.astype(jnp.int32), bpad])
  be = bfull[0:npk].reshape(1, npk)
  bo = bfull[npk:acc_rows].reshape(1, npk)

  degp = deg_call(dst3, zer64, ones64)
  degpk = degp.reshape(NC, npk, 128)
  dinv, g = tc0(degpk, x, params['Ws'][0])
  hs = []
  for l in range(nl):
    g64 = g.reshape(acc_rows, h)
    sp = scat_call(g64, src3, dst3, zer64)
    sp_pk = sp.reshape(NC, npk, 128)
    if l < nl - 1:
      hnew, g = tcmid(dinv, sp_pk, b_pk[l], w2s[l])
      hs.append(hnew)
    else:
      hs.append(tclast(dinv, sp_pk, b_pk[l]))
  return tcfin(*hs, *wjk2, bjk_pk, be, bo,
               params['Wl1'], params['bl1'].reshape(1, h),
               params['Wl2'], params['bl2'].reshape(1, ncls))


# speedup vs baseline: 1.0638x; 1.0638x over previous
import functools

import jax
import jax.numpy as jnp
from jax import lax
from jax.experimental import pallas as pl
from jax.experimental.pallas import tpu as pltpu
from jax.experimental.pallas import tpu_sc as plsc

NC = 2
NS = 16
NW = NC * NS
CHUNK = 128
NBUF = 8

_mesh = plsc.VectorSubcoreMesh(core_axis_name="c", subcore_axis_name="s")
_sc_params = pltpu.CompilerParams(use_tc_tiling_on_sc=False)


def _deg_body(dst3, zer, ones_h, out, acc, dst_v, ones_v, ss, *, cpw, rpt):
  c = lax.axis_index("c")
  s = lax.axis_index("s")
  wid = c * NS + s
  pltpu.sync_copy(zer, acc.at[pl.ds(s * rpt, rpt)])
  pltpu.sync_copy(dst3.at[wid], dst_v)
  pltpu.sync_copy(ones_h, ones_v)
  plsc.subcore_barrier()

  def body(k, carry):
    base = k * NBUF
    for b in range(NBUF):
      j = base + b
      pltpu.async_copy(ones_v, acc.at[dst_v.at[j]], ss, add=True)

      @pl.when(j >= 4)
      def _():
        pltpu.make_async_copy(ones_v, acc.at[dst_v.at[0]], ss).wait()

    return carry

  lax.fori_loop(0, cpw // NBUF, body, 0)
  for _ in range(4):
    pltpu.make_async_copy(ones_v, acc.at[dst_v.at[0]], ss).wait()
  plsc.subcore_barrier()
  pltpu.sync_copy(acc.at[pl.ds(s * rpt, rpt)], out.at[c, pl.ds(s * rpt, rpt)])


def _scat_body(g, src3, dst3, zer, out, acc, gsh, src_v, dst_v, rows_v, sg, ss,
               *, cpw, rpt):
  c = lax.axis_index("c")
  s = lax.axis_index("s")
  wid = c * NS + s
  pltpu.async_copy(g.at[pl.ds(s * rpt, rpt)], gsh.at[pl.ds(s * rpt, rpt)], sg)

  @pl.when(c == 0)
  def _():
    pltpu.async_copy(g.at[pl.ds(s * rpt, rpt)], acc.at[pl.ds(s * rpt, rpt)],
                     sg)

  @pl.when(c != 0)
  def _():
    pltpu.async_copy(zer, acc.at[pl.ds(s * rpt, rpt)], sg)

  pltpu.make_async_copy(g.at[pl.ds(s * rpt, rpt)], gsh.at[pl.ds(s * rpt, rpt)],
                        sg).wait()
  pltpu.make_async_copy(zer, acc.at[pl.ds(s * rpt, rpt)], sg).wait()
  plsc.subcore_barrier()

  rows = rows_v
  rnb = len(rows)
  ahead = 2
  depth = rnb - ahead
  hw = cpw // 2

  for phase in range(2):
    pltpu.sync_copy(src3.at[wid, pl.ds(phase * hw, hw)], src_v)
    pltpu.sync_copy(dst3.at[wid, pl.ds(phase * hw, hw)], dst_v)
    for b in range(ahead):
      pltpu.async_copy(gsh.at[src_v.at[b]], rows[b], sg)

    def body(k, carry):
      base = k * rnb
      for b in range(rnb):
        j = base + b

        @pl.when(j >= depth)
        def _():
          pltpu.make_async_copy(rows[b], acc.at[dst_v.at[0]], ss).wait()

        pltpu.make_async_copy(gsh.at[src_v.at[0]], rows[b], sg).wait()
        pltpu.async_copy(rows[b], acc.at[dst_v.at[j]], ss, add=True)

        @pl.when(j + ahead < hw)
        def _():
          pltpu.async_copy(gsh.at[src_v.at[j + ahead]],
                           rows[(b + ahead) % rnb], sg)

      return carry

    lax.fori_loop(0, hw // rnb, body, 0)
    for b in range(depth):
      pltpu.make_async_copy(rows[b], acc.at[dst_v.at[0]], ss).wait()
  plsc.subcore_barrier()
  pltpu.sync_copy(acc.at[pl.ds(s * rpt, rpt)], out.at[c, pl.ds(s * rpt, rpt)])


def _tc0_body(degpk, x, w0, dinv_out, g0):
  npk = dinv_out.shape[0]
  n = x.shape[0]
  dinv_pk = lax.rsqrt(degpk[0] + degpk[1] + 1.0)
  dinv_out[...] = dinv_pk
  xv = x[...]
  xa = xv[0:npk]
  xb = jnp.concatenate(
      [xv[npk:n], jnp.zeros((2 * npk - n, x.shape[1]), jnp.float32)], axis=0)
  t0 = jnp.concatenate(
      [jnp.dot(xa, w0[...], preferred_element_type=jnp.float32),
       jnp.dot(xb, w0[...], preferred_element_type=jnp.float32)], axis=1)
  g0[...] = dinv_pk * t0


def _tcmid_body(dinv, sp, b, w2, h_out, g_out):
  h = jnp.maximum(dinv[...] * (sp[0] + sp[1]) + b[...], 0.0)
  h_out[...] = h
  g_out[...] = dinv[...] * jnp.dot(h, w2[...],
                                   preferred_element_type=jnp.float32)


def _tclast_body(dinv, sp, b, h_out):
  h_out[...] = jnp.maximum(dinv[...] * (sp[0] + sp[1]) + b[...], 0.0)


def _tcfin_body(h1, h2, h3, h4, h5, h6, wjk1, wjk2, wjk3, wjk4, wjk5, wjk6,
                bjk, be, bo, wl1, bl1, wl2, bl2, out):
  hs = (h1, h2, h3, h4, h5, h6)
  ws = (wjk1, wjk2, wjk3, wjk4, wjk5, wjk6)
  acc = None
  for h, w in zip(hs, ws):
    t = jnp.dot(h[...], w[...], preferred_element_type=jnp.float32)
    acc = t if acc is None else acc + t
  hjk = jnp.maximum(acc + bjk[...], 0.0)
  ng = out.shape[0]
  npk = be.shape[1]
  gids = lax.broadcasted_iota(jnp.int32, (ng, npk), 0)
  ohe = jnp.where(gids == be[...], 1.0, 0.0).astype(jnp.float32)
  oho = jnp.where(gids == bo[...], 1.0, 0.0).astype(jnp.float32)
  pooled = (jnp.dot(ohe, hjk[:, 0:64], preferred_element_type=jnp.float32) +
            jnp.dot(oho, hjk[:, 64:128], preferred_element_type=jnp.float32))
  p1 = jnp.maximum(
      jnp.dot(pooled, wl1[...], preferred_element_type=jnp.float32) + bl1[...],
      0.0)
  logits = jnp.dot(p1, wl2[...], preferred_element_type=jnp.float32) + bl2[...]
  m = jnp.max(logits, axis=1, keepdims=True)
  e = jnp.exp(logits - m)
  out[...] = e / jnp.sum(e, axis=1, keepdims=True)


def _blockdiag2(w):
  a, b = w.shape
  z = jnp.zeros((a, b), jnp.float32)
  return jnp.concatenate([jnp.concatenate([w, z], axis=1),
                          jnp.concatenate([z, w], axis=1)], axis=0)


def kernel(x, edge_index, batch, params):
  n = x.shape[0]
  e = edge_index.shape[1]
  h = params['Ws'][0].shape[1]
  nl = len(params['Ws'])
  ng = 64
  ncls = params['Wl2'].shape[1]

  rpt = -(-(n + 1) // (NS * 8)) * 8
  acc_rows = NS * rpt
  npk = acc_rows // 2
  cpw = -(-e // (NW * CHUNK))
  cpw = -(-cpw // NBUF) * NBUF
  e_pad = NW * cpw * CHUNK

  def remap(v):
    return jnp.where(v < npk, 2 * v, 2 * (v - npk) + 1)

  src = remap(edge_index[0])
  dst = remap(edge_index[1])
  junk = 2 * (n - npk) + 1
  pad = e_pad - e
  src3 = jnp.concatenate([src, jnp.zeros((pad,), jnp.int32)]).reshape(
      NW, cpw, CHUNK)
  dst3 = jnp.concatenate([dst, jnp.full((pad,), junk, jnp.int32)]).reshape(
      NW, cpw, CHUNK)
  ones64 = jnp.ones((CHUNK, h), jnp.float32)
  zer64 = jnp.zeros((rpt, h), jnp.float32)

  deg_call = pl.kernel(
      functools.partial(_deg_body, cpw=cpw, rpt=rpt),
      out_type=jax.ShapeDtypeStruct((NC, acc_rows, h), jnp.float32),
      mesh=_mesh,
      scratch_types=[
          pltpu.VMEM_SHARED((acc_rows, h), jnp.float32),
          pltpu.VMEM((cpw, CHUNK), jnp.int32),
          pltpu.VMEM((CHUNK, h), jnp.float32),
          pltpu.SemaphoreType.DMA,
      ],
      compiler_params=_sc_params,
  )
  scat_call = pl.kernel(
      functools.partial(_scat_body, cpw=cpw, rpt=rpt),
      out_type=jax.ShapeDtypeStruct((NC, acc_rows, h), jnp.float32),
      mesh=_mesh,
      scratch_types=[
          pltpu.VMEM_SHARED((acc_rows, h), jnp.float32),
          pltpu.VMEM_SHARED((acc_rows, h), jnp.float32),
          pltpu.VMEM((cpw // 2, CHUNK), jnp.int32),
          pltpu.VMEM((cpw // 2, CHUNK), jnp.int32),
          [pltpu.VMEM((CHUNK, h), jnp.float32) for _ in range(4)],
          pltpu.SemaphoreType.DMA,
          pltpu.SemaphoreType.DMA,
      ],
      compiler_params=_sc_params,
  )

  tc0 = pl.pallas_call(
      _tc0_body,
      out_shape=(jax.ShapeDtypeStruct((npk, 128), jnp.float32),
                 jax.ShapeDtypeStruct((npk, 128), jnp.float32)))
  tcmid = pl.pallas_call(
      _tcmid_body,
      out_shape=(jax.ShapeDtypeStruct((npk, 128), jnp.float32),
                 jax.ShapeDtypeStruct((npk, 128), jnp.float32)))
  tclast = pl.pallas_call(
      _tclast_body, out_shape=jax.ShapeDtypeStruct((npk, 128), jnp.float32))
  tcfin = pl.pallas_call(
      _tcfin_body, out_shape=jax.ShapeDtypeStruct((ng, ncls), jnp.float32))

  w2s = [_blockdiag2(w) for w in params['Ws'][1:]]
  wjk2 = [_blockdiag2(params['Wjk'][i * h:(i + 1) * h, :]) for i in range(nl)]
  b_pk = [jnp.tile(b, 2).reshape(1, 2 * h) for b in params['bs']]
  bjk_pk = jnp.tile(params['bjk'], 2).reshape(1, 2 * h)
  bpad = jnp.full((acc_rows - n,), -1, jnp.int32)
  bfull = jnp.concatenate([batch.astype(jnp.int32), bpad])
  be = bfull[0:npk].reshape(1, npk)
  bo = bfull[npk:acc_rows].reshape(1, npk)

  degp = deg_call(dst3, zer64, ones64)
  degpk = degp.reshape(NC, npk, 128)
  dinv, g = tc0(degpk, x, params['Ws'][0])
  hs = []
  for l in range(nl):
    g64 = g.reshape(acc_rows, h)
    sp = scat_call(g64, src3, dst3, zer64)
    sp_pk = sp.reshape(NC, npk, 128)
    if l < nl - 1:
      hnew, g = tcmid(dinv, sp_pk, b_pk[l], w2s[l])
      hs.append(hnew)
    else:
      hs.append(tclast(dinv, sp_pk, b_pk[l]))
  return tcfin(*hs, *wjk2, bjk_pk, be, bo,
               params['Wl1'], params['bl1'].reshape(1, h),
               params['Wl2'], params['bl2'].reshape(1, ncls))
